# baseline (speedup 1.0000x reference)
import functools
import math

import jax
import jax.numpy as jnp
import numpy as np
from jax import lax
from jax.experimental import pallas as pl
from jax.experimental.pallas import tpu as pltpu
from jax.experimental.pallas import tpu_sc as plsc

_D_MODEL = 100000
_PADW = 100096
_DIM = 64
_BATCH = 4096
_SEQ = 200

_NC = 2
_NS = 16
_NW = _NC * _NS
_CPW = _DIM // _NW
_LANES = 16
_UNROLL = 32
_NIB = 4
_NOB = 2

_SCALE = np.float32(math.sqrt(_D_MODEL))


def _scale_pad_block(w_ref, o_ref):
    o_ref[:, pl.ds(0, _D_MODEL)] = w_ref[...] * _SCALE


def _scale_table(wt):
    return pl.pallas_call(
        _scale_pad_block,
        out_shape=jax.ShapeDtypeStruct((_DIM, _PADW), jnp.float32),
        grid=(_DIM // 8,),
        in_specs=[pl.BlockSpec((8, _D_MODEL), lambda i: (i, 0))],
        out_specs=pl.BlockSpec((8, _PADW), lambda i: (i, 0)),
    )(wt)


_sc_mesh = plsc.VectorSubcoreMesh(core_axis_name="c", subcore_axis_name="s")


@functools.partial(
    pl.kernel,
    mesh=_sc_mesh,
    out_type=jax.ShapeDtypeStruct((_SEQ, _DIM // 8, _BATCH // 128, 8, 128),
                                  jnp.float32),
    scratch_types=[
        pltpu.VMEM((_PADW // 128, 128), jnp.float32),
        pltpu.VMEM((_NIB, _BATCH // 128, 128), jnp.int32),
        pltpu.VMEM((_NOB, _BATCH // 128, 128), jnp.float32),
    ] + [pltpu.SemaphoreType.DMA] * (_NIB + _NOB),
    compiler_params=pltpu.CompilerParams(
        use_tc_tiling_on_sc=False, needs_layout_passes=False),
)
def _sc_colgather(ws_hbm, xt_hbm, out_hbm, col_v, idx_v, out_v, *sems):
    isem = sems[:_NIB]
    osem = sems[_NIB:]
    wid = lax.axis_index("s") * _NC + lax.axis_index("c")

    def prefetch_idx(s, sl):
        pltpu.async_copy(xt_hbm.at[s // 8, :, s % 8, :], idx_v.at[sl],
                         isem[sl])

    def wait_idx(sl):
        pltpu.make_async_copy(xt_hbm.at[0, :, 0, :], idx_v.at[sl],
                              isem[sl]).wait()

    def issue_out(s, c, sl):
        pltpu.async_copy(out_v.at[sl], out_hbm.at[s, c // 8, :, c % 8],
                         osem[sl])

    def wait_out(sl):
        pltpu.make_async_copy(out_v.at[sl], out_hbm.at[0, 0, :, 0],
                              osem[sl]).wait()

    def gather_row(isl, osl):
        @plsc.parallel_loop(0, _BATCH, _LANES, unroll=_UNROLL)
        def _(off):
            iv = idx_v[isl, off // 128, pl.ds(off % 128, _LANES)]
            out_v[osl, off // 128, pl.ds(off % 128, _LANES)] = (
                plsc.load_gather(col_v, [iv >> 7, iv & 127]))

    for ci in range(_CPW):
        c = wid * _CPW + ci
        pltpu.sync_copy(ws_hbm.at[c // 8, :, c % 8, :], col_v)
        for p in range(_NIB - 1):
            prefetch_idx(p, p)

        def sbody(g, carry):
            for b in range(4):
                s = g * 4 + b
                psl = (b + _NIB - 1) % _NIB
                if b == 0:
                    prefetch_idx(s + _NIB - 1, psl)
                else:
                    @pl.when(g < _SEQ // 4 - 1)
                    def _():
                        prefetch_idx(s + _NIB - 1, psl)
                wait_idx(b)
                if b >= _NOB:
                    wait_out(b % _NOB)
                else:
                    @pl.when(g >= 1)
                    def _():
                        wait_out(b % _NOB)
                gather_row(b, b % _NOB)
                issue_out(s, c, b % _NOB)
            return carry

        lax.fori_loop(0, _SEQ // 4, sbody, 0)
        wait_out(0)
        wait_out(1)


def kernel(x, W):
    ws = _scale_table(W.T)
    ws4 = ws.reshape(8, 8, _PADW // 128, 128).transpose(0, 2, 1, 3)
    xt4 = x.T.reshape(_SEQ // 8, 8, _BATCH // 128, 128).transpose(0, 2, 1, 3)
    out5 = _sc_colgather(ws4, xt4)
    t = jnp.transpose(out5, (2, 4, 0, 1, 3))
    return t.reshape(_BATCH, _SEQ, _DIM)


# speedup vs baseline: 1.1448x; 1.1448x over previous
import functools
import math

import jax
import jax.numpy as jnp
import numpy as np
from jax import lax
from jax.experimental import pallas as pl
from jax.experimental.pallas import tpu as pltpu
from jax.experimental.pallas import tpu_sc as plsc

_D_MODEL = 100000
_PADW = 100096
_DIM = 64
_BATCH = 4096
_SEQ = 200

_NC = 2
_NS = 16
_NW = _NC * _NS
_CPW = _DIM // _NW
_LANES = 16
_UNROLL = 16
_NIB = 4
_NOB = 2

_SCALE = np.float32(math.sqrt(_D_MODEL))


def _scale_pad_block(w_ref, o_ref):
    o_ref[:, pl.ds(0, _D_MODEL)] = w_ref[...] * _SCALE


def _scale_table(wt):
    return pl.pallas_call(
        _scale_pad_block,
        out_shape=jax.ShapeDtypeStruct((_DIM, _PADW), jnp.float32),
        grid=(_DIM // 8,),
        in_specs=[pl.BlockSpec((8, _D_MODEL), lambda i: (i, 0))],
        out_specs=pl.BlockSpec((8, _PADW), lambda i: (i, 0)),
    )(wt)


_sc_mesh = plsc.VectorSubcoreMesh(core_axis_name="c", subcore_axis_name="s")


@functools.partial(
    pl.kernel,
    mesh=_sc_mesh,
    out_type=jax.ShapeDtypeStruct((_SEQ, _DIM // 8, _BATCH // 128, 8, 128),
                                  jnp.float32),
    scratch_types=[
        pltpu.VMEM((_PADW // 128, 128), jnp.float32),
        pltpu.VMEM((_NIB, _BATCH // 128, 128), jnp.int32),
        pltpu.VMEM((_NOB, _BATCH // 128, 128), jnp.float32),
        pltpu.VMEM_SHARED((2, _BATCH // 128, 8, 128), jnp.int32),
    ] + [pltpu.SemaphoreType.DMA] * (_NIB + _NOB + 1),
    compiler_params=pltpu.CompilerParams(
        use_tc_tiling_on_sc=False, needs_layout_passes=False),
)
def _sc_colgather(ws_hbm, xt_hbm, out_hbm, col_v, idx_v, out_v, xsp, *sems):
    isem = sems[:_NIB]
    osem = sems[_NIB:_NIB + _NOB]
    bsem = sems[_NIB + _NOB]
    wid = lax.axis_index("s") * _NC + lax.axis_index("c")
    lead = lax.axis_index("s") == 0
    nblk = _SEQ // 8

    def prefetch_idx(r, sl, buf):
        pltpu.async_copy(xsp.at[buf, :, r, :], idx_v.at[sl], isem[sl])

    def wait_idx(sl):
        pltpu.make_async_copy(xsp.at[0, :, 0, :], idx_v.at[sl],
                              isem[sl]).wait()

    def issue_out(s, c, sl):
        pltpu.async_copy(out_v.at[sl], out_hbm.at[s, c // 8, :, c % 8],
                         osem[sl])

    def wait_out(sl):
        pltpu.make_async_copy(out_v.at[sl], out_hbm.at[0, 0, :, 0],
                              osem[sl]).wait()

    def stage_block(k, buf):
        pltpu.async_copy(xt_hbm.at[k], xsp.at[buf], bsem)

    def wait_block():
        pltpu.make_async_copy(xt_hbm.at[0], xsp.at[0], bsem).wait()

    def gather_row(isl, osl):
        @plsc.parallel_loop(0, _BATCH, _LANES, unroll=_UNROLL)
        def _(off):
            iv = idx_v[isl, off // 128, pl.ds(off % 128, _LANES)]
            out_v[osl, off // 128, pl.ds(off % 128, _LANES)] = (
                plsc.load_gather(col_v, [iv >> 7, iv & 127]))

    for ci in range(_CPW):
        c = wid * _CPW + ci
        pltpu.sync_copy(ws_hbm.at[c // 8, :, c % 8, :], col_v)

        @pl.when(lead)
        def _():
            stage_block(0, 0)
            wait_block()
        plsc.subcore_barrier()

        def blkbody(blk, carry):
            buf = lax.rem(blk, 2)

            @pl.when(jnp.logical_and(lead, blk < nblk - 1))
            def _():
                stage_block(blk + 1, 1 - buf)

            for p in range(_NIB - 1):
                prefetch_idx(p, p, buf)
            for b in range(8):
                s = blk * 8 + b
                if b + _NIB - 1 < 8:
                    prefetch_idx(b + _NIB - 1, (b + _NIB - 1) % _NIB, buf)
                wait_idx(b % _NIB)
                if b >= _NOB:
                    wait_out(b % _NOB)
                else:
                    @pl.when(blk >= 1)
                    def _():
                        wait_out(b % _NOB)
                gather_row(b % _NIB, b % _NOB)
                issue_out(s, c, b % _NOB)

            @pl.when(jnp.logical_and(lead, blk < nblk - 1))
            def _():
                wait_block()
            plsc.subcore_barrier()
            return carry

        lax.fori_loop(0, nblk, blkbody, 0)
        wait_out(0)
        wait_out(1)


def kernel(x, W):
    ws = _scale_table(W.T)
    ws4 = ws.reshape(8, 8, _PADW // 128, 128).transpose(0, 2, 1, 3)
    xt4 = x.T.reshape(_SEQ // 8, 8, _BATCH // 128, 128).transpose(0, 2, 1, 3)
    out5 = _sc_colgather(ws4, xt4)
    t = jnp.transpose(out5, (2, 4, 0, 1, 3))
    return t.reshape(_BATCH, _SEQ, _DIM)
